# Initial kernel scaffold; baseline (speedup 1.0000x reference)
import jax
import jax.numpy as jnp
from jax.experimental import pallas as pl


def kernel(x, params):
    raise NotImplementedError("write your pallas kernel here")


# speedup vs baseline: 1.2135x; 1.2135x over previous
import jax
import jax.numpy as jnp
from jax.experimental import pallas as pl
from jax.experimental.pallas import tpu as pltpu

N, D = 197, 768
H, HD, MLP = 12, 64, 3072
RH, LRANK, RES, EPS = 512, 256, 1, 1e-5
TBLK = 512

_HI = jax.lax.Precision.HIGHEST
_BF = jnp.bfloat16
_F32 = jnp.float32
_NT = (((1,), (1,)), ((), ()))


def _normalize(x):
    m = jnp.mean(x, axis=-1, keepdims=True)
    v = jnp.mean((x - m) ** 2, axis=-1, keepdims=True)
    return (x - m) / jnp.sqrt(v + EPS)


def _make_ra_kernel(bb):
    def _ra_kernel(x_ref, rg_ref, rb_ref, riw_ref, rib_ref, w1_ref,
                   b1_ref, w2_ref, b2_ref, w3_ref, b3_ref, g1_ref, bb1_ref,
                   wq_ref, wk_ref, wv_ref, wo_ref,
                   bq_ref, bk_ref, bv_ref, bo_ref,
                   h_ref, k0_ref, k1_ref):
        for b in range(bb):
            x = x_ref[b]
            xn = _normalize(x)
            z = xn * rg_ref[...] + rb_ref[...]
            z = jax.nn.gelu(jnp.dot(z.astype(_BF), riw_ref[...],
                                    preferred_element_type=_F32)
                            + rib_ref[...])
            g = jnp.mean(z, axis=0, keepdims=True)
            zc = jnp.concatenate([z, jnp.broadcast_to(g, z.shape)],
                                 axis=-1).astype(_BF)
            h1 = jax.nn.gelu(jnp.dot(zc, w1_ref[...],
                                     preferred_element_type=_F32)
                             + b1_ref[...])
            h2 = jax.nn.gelu(jnp.dot(h1.astype(_BF), w2_ref[...],
                                     preferred_element_type=_F32)
                             + b2_ref[...])
            lg = jnp.dot(h2.astype(_BF), w3_ref[...],
                         preferred_element_type=_F32) + b3_ref[...]
            row = jax.lax.broadcasted_iota(jnp.int32, (N, 1), 0)
            k0 = jnp.where(row < RES, 1.0,
                           (lg[:, 1:2] > lg[:, 0:1]).astype(_F32))
            k1 = jnp.where(row < RES, 1.0,
                           (lg[:, 3:4] > lg[:, 2:3]).astype(_F32))
            k0_ref[b] = k0
            k1_ref[b] = k1
            xln = (xn * g1_ref[...] + bb1_ref[...]).astype(_BF)
            q = ((jnp.dot(xln, wq_ref[...], preferred_element_type=_F32)
                  + bq_ref[...]) * 0.125).astype(_BF)
            k = (jnp.dot(xln, wk_ref[...], preferred_element_type=_F32)
                 + bk_ref[...]).astype(_BF)
            v = (jnp.dot(xln, wv_ref[...], preferred_element_type=_F32)
                 + bv_ref[...]).astype(_BF)
            es = []
            for h in range(H):
                sl = slice(h * HD, (h + 1) * HD)
                s = jax.lax.dot_general(q[:, sl], k[:, sl], _NT,
                                        preferred_element_type=_F32)
                es.append(jnp.exp(s))
            os = []
            for h in range(H):
                sl = slice(h * HD, (h + 1) * HD)
                rs = 1.0 / jnp.sum(es[h], axis=-1, keepdims=True)
                o = jnp.dot(es[h].astype(_BF), v[:, sl],
                            preferred_element_type=_F32) * rs
                os.append(o.astype(_BF))
            o_all = jnp.concatenate(os, axis=1)
            attn = jnp.dot(o_all, wo_ref[...],
                           preferred_element_type=_F32) + bo_ref[...]
            h_ref[b] = x + k0 * attn
    return _ra_kernel


def _ffn_kernel(h_ref, k0_ref, k1_ref, g2_ref, b2_ref, f1w_ref, f1b_ref,
                f2w_ref, f2b_ref, dcat_ref, ucat_ref, out_ref):
    hb = h_ref[...]
    hl = _normalize(hb) * g2_ref[...] + b2_ref[...]
    a = jax.nn.gelu(jnp.dot(hl.astype(_BF), f1w_ref[...],
                            preferred_element_type=_F32) + f1b_ref[...])
    f = jnp.dot(a.astype(_BF), f2w_ref[...],
                preferred_element_type=_F32) + f2b_ref[...]
    k0, k1 = k0_ref[...], k1_ref[...]
    out = hb + k1 * f
    ob = out.astype(_BF)
    m0 = (1.0 - k0) * (1.0 - k1)
    m1 = (1.0 - k0) * k1
    m2 = k0 * (1.0 - k1)
    mid = jnp.dot(ob, dcat_ref[...], preferred_element_type=_F32)
    msk = jnp.concatenate([jnp.broadcast_to(m0, (TBLK, LRANK)),
                           jnp.broadcast_to(m1, (TBLK, LRANK)),
                           jnp.broadcast_to(m2, (TBLK, LRANK))], axis=1)
    ap = jnp.dot((mid * msk).astype(_BF), ucat_ref[...],
                 preferred_element_type=_F32)
    out_ref[...] = out + ap


def _full(shape):
    nd = len(shape)
    return pl.BlockSpec(shape, lambda i: (0,) * nd)


def _row2(a):
    return a.reshape(1, -1)


def kernel(x, params):
    p = params
    B = x.shape[0]
    BB = 2 if B % 2 == 0 else 1

    h, k0, k1 = pl.pallas_call(
        _make_ra_kernel(BB),
        grid=(B // BB,),
        in_specs=[pl.BlockSpec((BB, N, D), lambda i: (i, 0, 0))]
        + [_full(s) for s in ((1, D), (1, D), (D, RH), (1, RH),
                              (2 * RH, RH), (1, RH), (RH, RH // 2),
                              (1, RH // 2), (RH // 2, 4), (1, 4),
                              (1, D), (1, D),
                              (D, D), (D, D), (D, D), (D, D),
                              (1, D), (1, D), (1, D), (1, D))],
        out_specs=[pl.BlockSpec((BB, N, D), lambda i: (i, 0, 0)),
                   pl.BlockSpec((BB, N, 1), lambda i: (i, 0, 0)),
                   pl.BlockSpec((BB, N, 1), lambda i: (i, 0, 0))],
        out_shape=[jax.ShapeDtypeStruct((B, N, D), _F32),
                   jax.ShapeDtypeStruct((B, N, 1), _F32),
                   jax.ShapeDtypeStruct((B, N, 1), _F32)],
    )(x, _row2(p['r_ln_g']), _row2(p['r_ln_b']), p['r_in_w'].astype(_BF),
      _row2(p['r_in_b']), p['r_o1_w'].astype(_BF),
      _row2(p['r_o1_b']), p['r_o2_w'].astype(_BF),
      _row2(p['r_o2_b']), p['r_o3_w'].astype(_BF), _row2(p['r_o3_b']),
      _row2(p['ln1_g']), _row2(p['ln1_b']),
      p['wq'].astype(_BF), p['wk'].astype(_BF), p['wv'].astype(_BF),
      p['wo'].astype(_BF), _row2(p['bq']), _row2(p['bk']),
      _row2(p['bv']), _row2(p['bo']))

    T = B * N
    TP = ((T + TBLK - 1) // TBLK) * TBLK
    pad = TP - T
    hf = jnp.pad(h.reshape(T, D), ((0, pad), (0, 0)))
    k0f = jnp.pad(k0.reshape(T, 1), ((0, pad), (0, 0)))
    k1f = jnp.pad(k1.reshape(T, 1), ((0, pad), (0, 0)))
    dcat = jnp.concatenate([p['lra0_down'], p['lra1_down'],
                            p['lra2_down']], axis=1).astype(_BF)
    ucat = jnp.concatenate([p['lra0_up'], p['lra1_up'],
                            p['lra2_up']], axis=0).astype(_BF)

    out = pl.pallas_call(
        _ffn_kernel,
        grid=(TP // TBLK,),
        in_specs=[pl.BlockSpec((TBLK, D), lambda i: (i, 0)),
                  pl.BlockSpec((TBLK, 1), lambda i: (i, 0)),
                  pl.BlockSpec((TBLK, 1), lambda i: (i, 0))]
        + [_full(s) for s in ((1, D), (1, D), (D, MLP), (1, MLP),
                              (MLP, D), (1, D),
                              (D, 3 * LRANK), (3 * LRANK, D))],
        out_specs=pl.BlockSpec((TBLK, D), lambda i: (i, 0)),
        out_shape=jax.ShapeDtypeStruct((TP, D), _F32),
    )(hf, k0f, k1f, _row2(p['ln2_g']), _row2(p['ln2_b']),
      p['fc1_w'].astype(_BF), _row2(p['fc1_b']),
      p['fc2_w'].astype(_BF), _row2(p['fc2_b']), dcat, ucat)

    return out[:T].reshape(B, N, D)
